# baseline (speedup 1.0000x reference)
import dataclasses
import functools
import math

import jax
import jax.numpy as jnp
from jax import lax
from jax.experimental import pallas as pl
from jax.experimental.pallas import tpu as pltpu
from jax.experimental.pallas import tpu_sc as plsc

D = 128
L = 16
NC = 2
NS = 16
NW = NC * NS
CHUNK = 128
INV_SQRT_D = 1.0 / math.sqrt(float(D))


def _prep_body(x_ref, w_ref, nw_ref):
    nw_ref[...] = jnp.dot(
        x_ref[...], w_ref[...],
        preferred_element_type=jnp.float32,
        precision=lax.Precision.HIGHEST,
    ) * INV_SQRT_D


def _node_transform(emb, w):
    n = emb.shape[0]
    return pl.pallas_call(
        _prep_body,
        out_shape=jax.ShapeDtypeStruct((n, D), jnp.float32),
    )(emb, w)


def _sc_scores(node_w, emb, src, tgt, bias16):
    n_edges = src.shape[0]
    n_nodes = emb.shape[0]
    assert n_edges % NW == 0
    per_w = n_edges // NW
    n_full = per_w // CHUNK
    tail = per_w - n_full * CHUNK
    assert n_full % 2 == 0 and tail % L == 0
    del n_nodes

    mesh = plsc.VectorSubcoreMesh(core_axis_name="c", subcore_axis_name="s")
    cp = pltpu.CompilerParams()
    if "needs_layout_passes" in pltpu.CompilerParams.__dataclass_fields__:
        cp = dataclasses.replace(cp, needs_layout_passes=False)

    @functools.partial(
        pl.kernel,
        mesh=mesh,
        compiler_params=cp,
        out_type=jax.ShapeDtypeStruct((n_edges,), jnp.float32),
        scratch_types=[
            pltpu.VMEM((per_w,), jnp.int32),
            pltpu.VMEM((per_w,), jnp.int32),
            pltpu.VMEM((CHUNK, D), jnp.float32),
            pltpu.VMEM((CHUNK, D), jnp.float32),
            pltpu.VMEM((CHUNK, D), jnp.float32),
            pltpu.VMEM((CHUNK, D), jnp.float32),
            pltpu.VMEM((CHUNK,), jnp.float32),
            pltpu.VMEM((CHUNK,), jnp.float32),
            pltpu.VMEM((L,), jnp.float32),
            pltpu.SemaphoreType.DMA,
            pltpu.SemaphoreType.DMA,
            pltpu.SemaphoreType.DMA,
            pltpu.SemaphoreType.DMA,
        ],
    )
    def k(nw_hbm, emb_hbm, src_hbm, tgt_hbm, b_hbm, out_hbm,
          si_all, ti_all, av0, bv0, av1, bv1, sv0, sv1, biasv,
          sa0, sb0, sa1, sb1):
        wid = lax.axis_index("s") * NC + lax.axis_index("c")
        base_w = wid * per_w
        pltpu.sync_copy(src_hbm.at[pl.ds(base_w, per_w)], si_all)
        pltpu.sync_copy(tgt_hbm.at[pl.ds(base_w, per_w)], ti_all)
        pltpu.sync_copy(b_hbm, biasv)
        bias_vec = biasv[pl.ds(0, L)] * INV_SQRT_D
        lane = lax.iota(jnp.int32, L)

        def fire(c, av_, bv_, sa, sb):
            off = c * CHUNK
            pltpu.async_copy(nw_hbm.at[si_all.at[pl.ds(off, CHUNK)]], av_, sa)
            pltpu.async_copy(emb_hbm.at[ti_all.at[pl.ds(off, CHUNK)]], bv_, sb)

        def drain(av_, bv_, sa, sb):
            pltpu.make_async_copy(
                nw_hbm.at[si_all.at[pl.ds(0, CHUNK)]], av_, sa).wait()
            pltpu.make_async_copy(
                emb_hbm.at[ti_all.at[pl.ds(0, CHUNK)]], bv_, sb).wait()

        IL = 4

        def compute_chunk(av_, bv_, sv_, n):
            @plsc.parallel_loop(0, n, step=L, unroll=2)
            def _(e0):
                vec = jnp.zeros((L,), jnp.float32)
                for t4 in range(0, L, IL):
                    accs = [None] * IL
                    for j in range(D // L):
                        loads = []
                        for t in range(IL):
                            e = e0 + t4 + t
                            loads.append((av_[e, pl.ds(j * L, L)],
                                          bv_[e, pl.ds(j * L, L)]))
                        for t in range(IL):
                            a, b = loads[t]
                            p = a * b
                            accs[t] = p if accs[t] is None else accs[t] + p
                    for t in range(IL):
                        vec = jnp.where(lane == t4 + t, jnp.sum(accs[t]), vec)
                sv_[pl.ds(e0, L)] = vec + bias_vec

        fire(0, av0, bv0, sa0, sb0)

        @pl.loop(0, n_full, step=2)
        def _(c):
            fire(c + 1, av1, bv1, sa1, sb1)
            drain(av0, bv0, sa0, sb0)
            compute_chunk(av0, bv0, sv0, CHUNK)
            pltpu.sync_copy(sv0, out_hbm.at[pl.ds(base_w + c * CHUNK, CHUNK)])

            @pl.when(c + 2 < n_full)
            def _():
                fire(c + 2, av0, bv0, sa0, sb0)

            drain(av1, bv1, sa1, sb1)
            compute_chunk(av1, bv1, sv1, CHUNK)
            pltpu.sync_copy(
                sv1, out_hbm.at[pl.ds(base_w + (c + 1) * CHUNK, CHUNK)])

        if tail:
            off_t = n_full * CHUNK
            ca = pltpu.async_copy(
                nw_hbm.at[si_all.at[pl.ds(off_t, tail)]],
                av0.at[pl.ds(0, tail)], sa0)
            cb = pltpu.async_copy(
                emb_hbm.at[ti_all.at[pl.ds(off_t, tail)]],
                bv0.at[pl.ds(0, tail)], sb0)
            ca.wait()
            cb.wait()
            compute_chunk(av0, bv0, sv0, tail)
            pltpu.sync_copy(
                sv0.at[pl.ds(0, tail)],
                out_hbm.at[pl.ds(base_w + off_t, tail)])

    return k(node_w, emb, src, tgt, bias16)


def kernel(node_embeddings, edge_index, W, b):
    emb = node_embeddings.astype(jnp.float32)
    w = W[0].astype(jnp.float32)
    src = edge_index[0]
    tgt = edge_index[1]
    bias16 = jnp.broadcast_to(b.astype(jnp.float32), (L,))
    node_w = _node_transform(emb, w)
    return _sc_scores(node_w, emb, src, tgt, bias16)


# speedup vs baseline: 1.9798x; 1.9798x over previous
import dataclasses
import functools
import math

import jax
import jax.numpy as jnp
from jax import lax
from jax.experimental import pallas as pl
from jax.experimental.pallas import tpu as pltpu
from jax.experimental.pallas import tpu_sc as plsc

D = 128
L = 16
NC = 2
NS = 16
NW = NC * NS
CHUNK = 128
INV_SQRT_D = 1.0 / math.sqrt(float(D))


def _prep_body(x_ref, w_ref, nw_ref):
    nw_ref[...] = jnp.dot(
        x_ref[...], w_ref[...],
        preferred_element_type=jnp.float32,
        precision=lax.Precision.HIGHEST,
    ) * INV_SQRT_D


def _node_transform(emb, w):
    n = emb.shape[0]
    return pl.pallas_call(
        _prep_body,
        out_shape=jax.ShapeDtypeStruct((n, D), jnp.float32),
    )(emb, w)


def _sc_scores(node_w, emb, src, tgt, bias16):
    n_edges = src.shape[0]
    n_nodes = emb.shape[0]
    assert n_edges % NW == 0
    per_w = n_edges // NW
    n_full = per_w // CHUNK
    tail = per_w - n_full * CHUNK
    assert n_full % 2 == 0 and tail % L == 0
    del n_nodes

    mesh = plsc.VectorSubcoreMesh(core_axis_name="c", subcore_axis_name="s")
    cp = pltpu.CompilerParams()
    if "needs_layout_passes" in pltpu.CompilerParams.__dataclass_fields__:
        cp = dataclasses.replace(cp, needs_layout_passes=False)

    @functools.partial(
        pl.kernel,
        mesh=mesh,
        compiler_params=cp,
        out_type=jax.ShapeDtypeStruct((n_edges,), jnp.float32),
        scratch_types=[
            pltpu.VMEM((per_w,), jnp.int32),
            pltpu.VMEM((per_w,), jnp.int32),
            pltpu.VMEM((CHUNK, D), jnp.float32),
            pltpu.VMEM((CHUNK, D), jnp.float32),
            pltpu.VMEM((CHUNK, D), jnp.float32),
            pltpu.VMEM((CHUNK, D), jnp.float32),
            pltpu.VMEM((CHUNK + L,), jnp.float32),
            pltpu.VMEM((CHUNK + L,), jnp.float32),
            pltpu.VMEM((L,), jnp.float32),
            pltpu.SemaphoreType.DMA,
            pltpu.SemaphoreType.DMA,
            pltpu.SemaphoreType.DMA,
            pltpu.SemaphoreType.DMA,
        ],
    )
    def k(nw_hbm, emb_hbm, src_hbm, tgt_hbm, b_hbm, out_hbm,
          si_all, ti_all, av0, bv0, av1, bv1, sv0, sv1, biasv,
          sa0, sb0, sa1, sb1):
        wid = lax.axis_index("s") * NC + lax.axis_index("c")
        base_w = wid * per_w
        pltpu.sync_copy(src_hbm.at[pl.ds(base_w, per_w)], si_all)
        pltpu.sync_copy(tgt_hbm.at[pl.ds(base_w, per_w)], ti_all)
        pltpu.sync_copy(b_hbm, biasv)
        bias_vec = biasv[pl.ds(0, L)] * INV_SQRT_D
        lane = lax.iota(jnp.int32, L)

        def fire(c, av_, bv_, sa, sb):
            off = c * CHUNK
            pltpu.async_copy(nw_hbm.at[si_all.at[pl.ds(off, CHUNK)]], av_, sa)
            pltpu.async_copy(emb_hbm.at[ti_all.at[pl.ds(off, CHUNK)]], bv_, sb)

        def drain(av_, bv_, sa, sb):
            pltpu.make_async_copy(
                nw_hbm.at[si_all.at[pl.ds(0, CHUNK)]], av_, sa).wait()
            pltpu.make_async_copy(
                emb_hbm.at[ti_all.at[pl.ds(0, CHUNK)]], bv_, sb).wait()

        mask15 = lane == (L - 1)
        bias_per_lane = bias_vec * (1.0 / L)

        def compute_chunk(av_, bv_, sv_, n):
            @plsc.parallel_loop(0, n, step=1, unroll=2)
            def _(e):
                acc0 = av_[e, pl.ds(0, L)] * bv_[e, pl.ds(0, L)]
                acc1 = av_[e, pl.ds(L, L)] * bv_[e, pl.ds(L, L)]
                for j in range(2, D // L, 2):
                    acc0 = acc0 + (av_[e, pl.ds(j * L, L)]
                                   * bv_[e, pl.ds(j * L, L)])
                    acc1 = acc1 + (av_[e, pl.ds((j + 1) * L, L)]
                                   * bv_[e, pl.ds((j + 1) * L, L)])
                scanned = plsc.cumsum(acc0 + acc1 + bias_per_lane)
                plsc.store_compressed(sv_.at[pl.ds(e, L)], scanned, mask=mask15)

        fire(0, av0, bv0, sa0, sb0)

        @pl.loop(0, n_full, step=2)
        def _(c):
            fire(c + 1, av1, bv1, sa1, sb1)
            drain(av0, bv0, sa0, sb0)
            compute_chunk(av0, bv0, sv0, CHUNK)
            pltpu.sync_copy(sv0.at[pl.ds(0, CHUNK)],
                            out_hbm.at[pl.ds(base_w + c * CHUNK, CHUNK)])

            @pl.when(c + 2 < n_full)
            def _():
                fire(c + 2, av0, bv0, sa0, sb0)

            drain(av1, bv1, sa1, sb1)
            compute_chunk(av1, bv1, sv1, CHUNK)
            pltpu.sync_copy(
                sv1.at[pl.ds(0, CHUNK)],
                out_hbm.at[pl.ds(base_w + (c + 1) * CHUNK, CHUNK)])

        if tail:
            off_t = n_full * CHUNK
            ca = pltpu.async_copy(
                nw_hbm.at[si_all.at[pl.ds(off_t, tail)]],
                av0.at[pl.ds(0, tail)], sa0)
            cb = pltpu.async_copy(
                emb_hbm.at[ti_all.at[pl.ds(off_t, tail)]],
                bv0.at[pl.ds(0, tail)], sb0)
            ca.wait()
            cb.wait()
            compute_chunk(av0, bv0, sv0, tail)
            pltpu.sync_copy(
                sv0.at[pl.ds(0, tail)],
                out_hbm.at[pl.ds(base_w + off_t, tail)])

    return k(node_w, emb, src, tgt, bias16)


def kernel(node_embeddings, edge_index, W, b):
    emb = node_embeddings.astype(jnp.float32)
    w = W[0].astype(jnp.float32)
    src = edge_index[0]
    tgt = edge_index[1]
    bias16 = jnp.broadcast_to(b.astype(jnp.float32), (L,))
    node_w = _node_transform(emb, w)
    return _sc_scores(node_w, emb, src, tgt, bias16)


# speedup vs baseline: 1.9815x; 1.0009x over previous
import dataclasses
import functools
import math

import jax
import jax.numpy as jnp
from jax import lax
from jax.experimental import pallas as pl
from jax.experimental.pallas import tpu as pltpu
from jax.experimental.pallas import tpu_sc as plsc

D = 128
L = 16
NC = 2
NS = 16
NW = NC * NS
CHUNK = 128
INV_SQRT_D = 1.0 / math.sqrt(float(D))


def _prep_body(x_ref, w_ref, nw_ref):
    nw_ref[...] = jnp.dot(
        x_ref[...], w_ref[...],
        preferred_element_type=jnp.float32,
        precision=lax.Precision.HIGHEST,
    ) * INV_SQRT_D


def _node_transform(emb, w):
    n = emb.shape[0]
    return pl.pallas_call(
        _prep_body,
        out_shape=jax.ShapeDtypeStruct((n, D), jnp.float32),
    )(emb, w)


def _sc_scores(node_w, emb, src, tgt, bias16):
    n_edges = src.shape[0]
    n_nodes = emb.shape[0]
    assert n_edges % NW == 0
    per_w = n_edges // NW
    n_full = per_w // CHUNK
    tail = per_w - n_full * CHUNK
    assert n_full % 2 == 0 and tail % L == 0
    del n_nodes

    mesh = plsc.VectorSubcoreMesh(core_axis_name="c", subcore_axis_name="s")
    cp = pltpu.CompilerParams()
    if "needs_layout_passes" in pltpu.CompilerParams.__dataclass_fields__:
        cp = dataclasses.replace(cp, needs_layout_passes=False)

    @functools.partial(
        pl.kernel,
        mesh=mesh,
        compiler_params=cp,
        out_type=jax.ShapeDtypeStruct((n_edges,), jnp.float32),
        scratch_types=[
            pltpu.VMEM((per_w,), jnp.int32),
            pltpu.VMEM((per_w,), jnp.int32),
            pltpu.VMEM((CHUNK, D), jnp.float32),
            pltpu.VMEM((CHUNK, D), jnp.float32),
            pltpu.VMEM((CHUNK, D), jnp.float32),
            pltpu.VMEM((CHUNK, D), jnp.float32),
            pltpu.VMEM((CHUNK + L,), jnp.float32),
            pltpu.VMEM((CHUNK + L,), jnp.float32),
            pltpu.VMEM((L,), jnp.float32),
            pltpu.SemaphoreType.DMA,
            pltpu.SemaphoreType.DMA,
            pltpu.SemaphoreType.DMA,
            pltpu.SemaphoreType.DMA,
        ],
    )
    def k(nw_hbm, emb_hbm, src_hbm, tgt_hbm, b_hbm, out_hbm,
          si_all, ti_all, av0, bv0, av1, bv1, sv0, sv1, biasv,
          sa0, sb0, sa1, sb1):
        wid = lax.axis_index("s") * NC + lax.axis_index("c")
        base_w = wid * per_w
        pltpu.sync_copy(src_hbm.at[pl.ds(base_w, per_w)], si_all)
        pltpu.sync_copy(tgt_hbm.at[pl.ds(base_w, per_w)], ti_all)
        pltpu.sync_copy(b_hbm, biasv)
        bias_vec = biasv[pl.ds(0, L)] * INV_SQRT_D
        lane = lax.iota(jnp.int32, L)

        def fire(c, av_, bv_, sa, sb):
            off = c * CHUNK
            pltpu.async_copy(nw_hbm.at[si_all.at[pl.ds(off, CHUNK)]], av_, sa)
            pltpu.async_copy(emb_hbm.at[ti_all.at[pl.ds(off, CHUNK)]], bv_, sb)

        def drain(av_, bv_, sa, sb):
            pltpu.make_async_copy(
                nw_hbm.at[si_all.at[pl.ds(0, CHUNK)]], av_, sa).wait()
            pltpu.make_async_copy(
                emb_hbm.at[ti_all.at[pl.ds(0, CHUNK)]], bv_, sb).wait()

        mask15 = lane == (L - 1)
        bias_per_lane = bias_vec * (1.0 / L)

        def compute_chunk(av_, bv_, sv_, n):
            @plsc.parallel_loop(0, n, step=1, unroll=4)
            def _(e):
                acc0 = av_[e, pl.ds(0, L)] * bv_[e, pl.ds(0, L)]
                acc1 = av_[e, pl.ds(L, L)] * bv_[e, pl.ds(L, L)]
                for j in range(2, D // L, 2):
                    acc0 = acc0 + (av_[e, pl.ds(j * L, L)]
                                   * bv_[e, pl.ds(j * L, L)])
                    acc1 = acc1 + (av_[e, pl.ds((j + 1) * L, L)]
                                   * bv_[e, pl.ds((j + 1) * L, L)])
                scanned = plsc.cumsum(acc0 + acc1 + bias_per_lane)
                plsc.store_compressed(sv_.at[pl.ds(e, L)], scanned, mask=mask15)

        fire(0, av0, bv0, sa0, sb0)

        @pl.loop(0, n_full, step=2)
        def _(c):
            fire(c + 1, av1, bv1, sa1, sb1)
            drain(av0, bv0, sa0, sb0)
            compute_chunk(av0, bv0, sv0, CHUNK)
            pltpu.sync_copy(sv0.at[pl.ds(0, CHUNK)],
                            out_hbm.at[pl.ds(base_w + c * CHUNK, CHUNK)])

            @pl.when(c + 2 < n_full)
            def _():
                fire(c + 2, av0, bv0, sa0, sb0)

            drain(av1, bv1, sa1, sb1)
            compute_chunk(av1, bv1, sv1, CHUNK)
            pltpu.sync_copy(
                sv1.at[pl.ds(0, CHUNK)],
                out_hbm.at[pl.ds(base_w + (c + 1) * CHUNK, CHUNK)])

        if tail:
            off_t = n_full * CHUNK
            ca = pltpu.async_copy(
                nw_hbm.at[si_all.at[pl.ds(off_t, tail)]],
                av0.at[pl.ds(0, tail)], sa0)
            cb = pltpu.async_copy(
                emb_hbm.at[ti_all.at[pl.ds(off_t, tail)]],
                bv0.at[pl.ds(0, tail)], sb0)
            ca.wait()
            cb.wait()
            compute_chunk(av0, bv0, sv0, tail)
            pltpu.sync_copy(
                sv0.at[pl.ds(0, tail)],
                out_hbm.at[pl.ds(base_w + off_t, tail)])

    return k(node_w, emb, src, tgt, bias16)


def kernel(node_embeddings, edge_index, W, b):
    emb = node_embeddings.astype(jnp.float32)
    w = W[0].astype(jnp.float32)
    src = edge_index[0]
    tgt = edge_index[1]
    bias16 = jnp.broadcast_to(b.astype(jnp.float32), (L,))
    node_w = _node_transform(emb, w)
    return _sc_scores(node_w, emb, src, tgt, bias16)


# speedup vs baseline: 1.9828x; 1.0006x over previous
import dataclasses
import functools
import math

import jax
import jax.numpy as jnp
from jax import lax
from jax.experimental import pallas as pl
from jax.experimental.pallas import tpu as pltpu
from jax.experimental.pallas import tpu_sc as plsc

D = 128
L = 16
NC = 2
NS = 16
NW = NC * NS
CHUNK = 128
INV_SQRT_D = 1.0 / math.sqrt(float(D))


def _prep_body(x_ref, w_ref, nw_ref):
    nw_ref[...] = jnp.dot(
        x_ref[...], w_ref[...],
        preferred_element_type=jnp.float32,
        precision=lax.Precision.HIGHEST,
    ) * INV_SQRT_D


def _node_transform(emb, w):
    n = emb.shape[0]
    return pl.pallas_call(
        _prep_body,
        out_shape=jax.ShapeDtypeStruct((n, D), jnp.float32),
    )(emb, w)


def _sc_scores(node_w, emb, src, tgt, bias16):
    n_edges = src.shape[0]
    n_nodes = emb.shape[0]
    assert n_edges % NW == 0
    per_w = n_edges // NW
    n_full = per_w // CHUNK
    tail = per_w - n_full * CHUNK
    assert n_full % 2 == 0 and tail % L == 0
    del n_nodes

    mesh = plsc.VectorSubcoreMesh(core_axis_name="c", subcore_axis_name="s")
    cp = pltpu.CompilerParams()
    if "needs_layout_passes" in pltpu.CompilerParams.__dataclass_fields__:
        cp = dataclasses.replace(cp, needs_layout_passes=False)

    @functools.partial(
        pl.kernel,
        mesh=mesh,
        compiler_params=cp,
        out_type=jax.ShapeDtypeStruct((n_edges,), jnp.float32),
        scratch_types=[
            pltpu.VMEM((per_w,), jnp.int32),
            pltpu.VMEM((per_w,), jnp.int32),
            pltpu.VMEM((CHUNK, D), jnp.float32),
            pltpu.VMEM((CHUNK, D), jnp.float32),
            pltpu.VMEM((CHUNK, D), jnp.float32),
            pltpu.VMEM((CHUNK, D), jnp.float32),
            pltpu.VMEM((CHUNK + L,), jnp.float32),
            pltpu.VMEM((CHUNK + L,), jnp.float32),
            pltpu.VMEM((L,), jnp.float32),
            pltpu.SemaphoreType.DMA,
            pltpu.SemaphoreType.DMA,
            pltpu.SemaphoreType.DMA,
            pltpu.SemaphoreType.DMA,
        ],
    )
    def k(nw_hbm, emb_hbm, src_hbm, tgt_hbm, b_hbm, out_hbm,
          si_all, ti_all, av0, bv0, av1, bv1, sv0, sv1, biasv,
          sa0, sb0, sa1, sb1):
        wid = lax.axis_index("s") * NC + lax.axis_index("c")
        base_w = wid * per_w
        pltpu.sync_copy(src_hbm.at[pl.ds(base_w, per_w)], si_all)
        pltpu.sync_copy(tgt_hbm.at[pl.ds(base_w, per_w)], ti_all)
        pltpu.sync_copy(b_hbm, biasv)
        bias_vec = biasv[pl.ds(0, L)] * INV_SQRT_D
        lane = lax.iota(jnp.int32, L)

        def fire(c, av_, bv_, sa, sb):
            off = c * CHUNK
            pltpu.async_copy(nw_hbm.at[si_all.at[pl.ds(off, CHUNK)]], av_, sa)
            pltpu.async_copy(emb_hbm.at[ti_all.at[pl.ds(off, CHUNK)]], bv_, sb)

        def drain(av_, bv_, sa, sb):
            pltpu.make_async_copy(
                nw_hbm.at[si_all.at[pl.ds(0, CHUNK)]], av_, sa).wait()
            pltpu.make_async_copy(
                emb_hbm.at[ti_all.at[pl.ds(0, CHUNK)]], bv_, sb).wait()

        mask15 = lane == (L - 1)
        bias_per_lane = bias_vec * (1.0 / L)

        def compute_chunk(av_, bv_, sv_, n):
            @plsc.parallel_loop(0, n, step=1, unroll=2)
            def _(e):
                acc0 = av_[e, pl.ds(0, L)] * bv_[e, pl.ds(0, L)]
                acc1 = av_[e, pl.ds(L, L)] * bv_[e, pl.ds(L, L)]
                for j in range(2, D // L, 2):
                    acc0 = acc0 + (av_[e, pl.ds(j * L, L)]
                                   * bv_[e, pl.ds(j * L, L)])
                    acc1 = acc1 + (av_[e, pl.ds((j + 1) * L, L)]
                                   * bv_[e, pl.ds((j + 1) * L, L)])
                scanned = plsc.cumsum(acc0 + acc1 + bias_per_lane)
                plsc.store_compressed(sv_.at[pl.ds(e, L)], scanned, mask=mask15)

        fire(0, av0, bv0, sa0, sb0)

        @pl.loop(0, n_full, step=2)
        def _(c):
            fire(c + 1, av1, bv1, sa1, sb1)
            drain(av0, bv0, sa0, sb0)
            compute_chunk(av0, bv0, sv0, CHUNK)
            pltpu.sync_copy(sv0.at[pl.ds(0, CHUNK)],
                            out_hbm.at[pl.ds(base_w + c * CHUNK, CHUNK)])

            @pl.when(c + 2 < n_full)
            def _():
                fire(c + 2, av0, bv0, sa0, sb0)

            drain(av1, bv1, sa1, sb1)
            compute_chunk(av1, bv1, sv1, CHUNK)
            pltpu.sync_copy(
                sv1.at[pl.ds(0, CHUNK)],
                out_hbm.at[pl.ds(base_w + (c + 1) * CHUNK, CHUNK)])

        if tail:
            off_t = n_full * CHUNK
            ca = pltpu.async_copy(
                nw_hbm.at[si_all.at[pl.ds(off_t, tail)]],
                av0.at[pl.ds(0, tail)], sa0)
            cb = pltpu.async_copy(
                emb_hbm.at[ti_all.at[pl.ds(off_t, tail)]],
                bv0.at[pl.ds(0, tail)], sb0)
            ca.wait()
            cb.wait()
            compute_chunk(av0, bv0, sv0, tail)
            pltpu.sync_copy(
                sv0.at[pl.ds(0, tail)],
                out_hbm.at[pl.ds(base_w + off_t, tail)])

    return k(node_w, emb, src, tgt, bias16)


def kernel(node_embeddings, edge_index, W, b):
    emb = node_embeddings.astype(jnp.float32)
    w = W[0].astype(jnp.float32)
    src = edge_index[0]
    tgt = edge_index[1]
    bias16 = jnp.broadcast_to(b.astype(jnp.float32), (L,))
    node_w = _node_transform(emb, w)
    return _sc_scores(node_w, emb, src, tgt, bias16)


# speedup vs baseline: 2.2018x; 1.1105x over previous
import dataclasses
import functools
import math

import jax
import jax.numpy as jnp
from jax import lax
from jax.experimental import pallas as pl
from jax.experimental.pallas import tpu as pltpu
from jax.experimental.pallas import tpu_sc as plsc

D = 128
L = 16
NC = 2
NS = 16
NW = NC * NS
CHUNK = 128
INV_SQRT_D = 1.0 / math.sqrt(float(D))


def _prep_body(x_ref, w_ref, nw_ref):
    nw_ref[...] = jnp.dot(
        x_ref[...], w_ref[...],
        preferred_element_type=jnp.float32,
        precision=lax.Precision.HIGHEST,
    ) * INV_SQRT_D


def _node_transform(emb, w):
    n = emb.shape[0]
    return pl.pallas_call(
        _prep_body,
        out_shape=jax.ShapeDtypeStruct((n, D), jnp.float32),
    )(emb, w)


def _sc_scores(node_w, emb, src, tgt, bias16):
    n_edges = src.shape[0]
    n_nodes = emb.shape[0]
    assert n_edges % NW == 0
    per_w = n_edges // NW
    n_full = per_w // CHUNK
    tail = per_w - n_full * CHUNK
    assert n_full % 2 == 0 and tail % L == 0
    del n_nodes

    mesh = plsc.VectorSubcoreMesh(core_axis_name="c", subcore_axis_name="s")
    cp = pltpu.CompilerParams()
    if "needs_layout_passes" in pltpu.CompilerParams.__dataclass_fields__:
        cp = dataclasses.replace(cp, needs_layout_passes=False)

    @functools.partial(
        pl.kernel,
        mesh=mesh,
        compiler_params=cp,
        out_type=jax.ShapeDtypeStruct((n_edges,), jnp.float32),
        scratch_types=[
            pltpu.VMEM((per_w,), jnp.int32),
            pltpu.VMEM((per_w,), jnp.int32),
            pltpu.VMEM((CHUNK, D), jnp.float32),
            pltpu.VMEM((CHUNK, D), jnp.float32),
            pltpu.VMEM((CHUNK, D), jnp.float32),
            pltpu.VMEM((CHUNK, D), jnp.float32),
            pltpu.VMEM((CHUNK, D), jnp.float32),
            pltpu.VMEM((CHUNK, D), jnp.float32),
            pltpu.VMEM((CHUNK + L,), jnp.float32),
            pltpu.VMEM((CHUNK + L,), jnp.float32),
            pltpu.VMEM((CHUNK + L,), jnp.float32),
            pltpu.VMEM((L,), jnp.float32),
            pltpu.SemaphoreType.DMA,
            pltpu.SemaphoreType.DMA,
            pltpu.SemaphoreType.DMA,
            pltpu.SemaphoreType.DMA,
            pltpu.SemaphoreType.DMA,
            pltpu.SemaphoreType.DMA,
        ],
    )
    def k(nw_hbm, emb_hbm, src_hbm, tgt_hbm, b_hbm, out_hbm,
          si_all, ti_all, av0, bv0, av1, bv1, av2, bv2, sv0, sv1, sv2, biasv,
          sa0, sb0, sa1, sb1, sa2, sb2):
        wid = lax.axis_index("s") * NC + lax.axis_index("c")
        base_w = wid * per_w
        pltpu.sync_copy(src_hbm.at[pl.ds(base_w, per_w)], si_all)
        pltpu.sync_copy(tgt_hbm.at[pl.ds(base_w, per_w)], ti_all)
        pltpu.sync_copy(b_hbm, biasv)
        bias_vec = biasv[pl.ds(0, L)] * INV_SQRT_D
        lane = lax.iota(jnp.int32, L)

        def fire(c, av_, bv_, sa, sb):
            off = c * CHUNK
            pltpu.async_copy(nw_hbm.at[si_all.at[pl.ds(off, CHUNK)]], av_, sa)
            pltpu.async_copy(emb_hbm.at[ti_all.at[pl.ds(off, CHUNK)]], bv_, sb)

        def drain(av_, bv_, sa, sb):
            pltpu.make_async_copy(
                nw_hbm.at[si_all.at[pl.ds(0, CHUNK)]], av_, sa).wait()
            pltpu.make_async_copy(
                emb_hbm.at[ti_all.at[pl.ds(0, CHUNK)]], bv_, sb).wait()

        mask15 = lane == (L - 1)
        bias_per_lane = bias_vec * (1.0 / L)

        def compute_chunk(av_, bv_, sv_, n):
            @plsc.parallel_loop(0, n, step=1, unroll=2)
            def _(e):
                acc0 = av_[e, pl.ds(0, L)] * bv_[e, pl.ds(0, L)]
                acc1 = av_[e, pl.ds(L, L)] * bv_[e, pl.ds(L, L)]
                for j in range(2, D // L, 2):
                    acc0 = acc0 + (av_[e, pl.ds(j * L, L)]
                                   * bv_[e, pl.ds(j * L, L)])
                    acc1 = acc1 + (av_[e, pl.ds((j + 1) * L, L)]
                                   * bv_[e, pl.ds((j + 1) * L, L)])
                scanned = plsc.cumsum(acc0 + acc1 + bias_per_lane)
                plsc.store_compressed(sv_.at[pl.ds(e, L)], scanned, mask=mask15)

        assert n_full % 3 == 0
        fire(0, av0, bv0, sa0, sb0)
        fire(1, av1, bv1, sa1, sb1)

        @pl.loop(0, n_full, step=3)
        def _(c):
            bufs = ((av0, bv0, sv0, sa0, sb0),
                    (av1, bv1, sv1, sa1, sb1),
                    (av2, bv2, sv2, sa2, sb2))
            for p in range(3):
                av_, bv_, sv_, sa, sb = bufs[p]
                nav, nbv, _, nsa, nsb = bufs[(p + 2) % 3]
                drain(av_, bv_, sa, sb)

                @pl.when(c + p + 2 < n_full)
                def _():
                    fire(c + p + 2, nav, nbv, nsa, nsb)

                compute_chunk(av_, bv_, sv_, CHUNK)
                pltpu.sync_copy(
                    sv_.at[pl.ds(0, CHUNK)],
                    out_hbm.at[pl.ds(base_w + (c + p) * CHUNK, CHUNK)])

        if tail:
            off_t = n_full * CHUNK
            ca = pltpu.async_copy(
                nw_hbm.at[si_all.at[pl.ds(off_t, tail)]],
                av0.at[pl.ds(0, tail)], sa0)
            cb = pltpu.async_copy(
                emb_hbm.at[ti_all.at[pl.ds(off_t, tail)]],
                bv0.at[pl.ds(0, tail)], sb0)
            ca.wait()
            cb.wait()
            compute_chunk(av0, bv0, sv0, tail)
            pltpu.sync_copy(
                sv0.at[pl.ds(0, tail)],
                out_hbm.at[pl.ds(base_w + off_t, tail)])

    return k(node_w, emb, src, tgt, bias16)


def kernel(node_embeddings, edge_index, W, b):
    emb = node_embeddings.astype(jnp.float32)
    w = W[0].astype(jnp.float32)
    src = edge_index[0]
    tgt = edge_index[1]
    bias16 = jnp.broadcast_to(b.astype(jnp.float32), (L,))
    node_w = _node_transform(emb, w)
    return _sc_scores(node_w, emb, src, tgt, bias16)
